# baseline (speedup 1.0000x reference)
import functools

import jax
import jax.numpy as jnp
from jax import lax
from jax.experimental import pallas as pl
from jax.experimental.pallas import tpu as pltpu
from jax.experimental.pallas import tpu_sc as plsc

_INFO = plsc.get_sparse_core_info()
_NC = _INFO.num_cores
_NS = _INFO.num_subcores
_L = _INFO.num_lanes
_NW = _NC * _NS

_CH = 32768


def kernel(x, pos_table):
    B, S, D = x.shape
    N = S * D
    PW = N // _NW
    NCH = PW // _CH
    xf = x.reshape(B * N)
    pf = pos_table.reshape(-1)[:N]

    @functools.partial(
        pl.kernel,
        mesh=plsc.VectorSubcoreMesh(core_axis_name="c", subcore_axis_name="s"),
        out_type=jax.ShapeDtypeStruct((B * N,), jnp.float32),
        scratch_types=[
            pltpu.VMEM((_CH,), jnp.float32),
            pltpu.VMEM((_CH,), jnp.float32),
        ],
    )
    def run(x_hbm, pos_hbm, out_hbm, pos_v, xb_v):
        wid = lax.axis_index("s") * _NC + lax.axis_index("c")
        base = wid * PW
        for c in range(NCH):
            off = base + c * _CH
            pltpu.sync_copy(pos_hbm.at[pl.ds(off, _CH)], pos_v)
            for b in range(B):
                xoff = b * N + off
                pltpu.sync_copy(x_hbm.at[pl.ds(xoff, _CH)], xb_v)

                def add_body(i, carry):
                    sl = pl.ds(i * _L, _L)
                    xb_v[sl] = xb_v[sl] + pos_v[sl]
                    return carry

                lax.fori_loop(0, _CH // _L, add_body, 0)
                pltpu.sync_copy(xb_v, out_hbm.at[pl.ds(xoff, _CH)])

    return run(xf, pf).reshape(B, S, D)


# speedup vs baseline: 1.8111x; 1.8111x over previous
import functools

import jax
import jax.numpy as jnp
from jax import lax
from jax.experimental import pallas as pl
from jax.experimental.pallas import tpu as pltpu
from jax.experimental.pallas import tpu_sc as plsc

_INFO = plsc.get_sparse_core_info()
_NC = _INFO.num_cores
_NS = _INFO.num_subcores
_L = _INFO.num_lanes
_NW = _NC * _NS

_CH = 16384
_NBUF = 5


def kernel(x, pos_table):
    B, S, D = x.shape
    N = S * D
    PW = N // _NW
    NCH = PW // _CH
    NX = NCH * B
    xf = x.reshape(B * N)
    pf = pos_table.reshape(-1)[:N]

    @functools.partial(
        pl.kernel,
        mesh=plsc.VectorSubcoreMesh(core_axis_name="c", subcore_axis_name="s"),
        out_type=jax.ShapeDtypeStruct((B * N,), jnp.float32),
        scratch_types=(
            [pltpu.VMEM((_CH,), jnp.float32) for _ in range(2)]
            + [pltpu.VMEM((_CH,), jnp.float32) for _ in range(_NBUF)]
            + [
                pltpu.SemaphoreType.DMA((2,)),
                pltpu.SemaphoreType.DMA((_NBUF,)),
                pltpu.SemaphoreType.DMA((_NBUF,)),
            ]
        ),
    )
    def run(x_hbm, pos_hbm, out_hbm, p0, p1, *rest):
        xbufs = list(rest[:_NBUF])
        sp, sl, ss = rest[_NBUF:]
        pbufs = [p0, p1]
        wid = lax.axis_index("s") * _NC + lax.axis_index("c")
        base = wid * PW

        def x_slice(k):
            c, b = divmod(k, B)
            return pl.ds(b * N + base + c * _CH, _CH)

        pos_d, load_d, store_d = {}, {}, {}
        for c in range(min(2, NCH)):
            pos_d[c] = pltpu.async_copy(
                pos_hbm.at[pl.ds(base + c * _CH, _CH)], pbufs[c], sp.at[c])
        for k in range(min(_NBUF - 1, NX)):
            load_d[k] = pltpu.async_copy(
                x_hbm.at[x_slice(k)], xbufs[k % _NBUF], sl.at[k % _NBUF])

        for k in range(NX):
            c, b = divmod(k, B)
            j = k % _NBUF
            if b == 0:
                pos_d[c].wait()
            load_d[k].wait()
            pb, xb = pbufs[c % 2], xbufs[j]

            @plsc.parallel_loop(0, _CH // _L, unroll=8)
            def add_body(i, pb=pb, xb=xb):
                sl_ = pl.ds(i * _L, _L)
                plsc.addupdate(xb.at[sl_], pb[sl_])

            store_d[k] = pltpu.async_copy(xb, out_hbm.at[x_slice(k)], ss.at[j])
            nk = k + _NBUF - 1
            if nk < NX:
                jj = nk % _NBUF
                if nk - _NBUF >= 0:
                    store_d[nk - _NBUF].wait()
                load_d[nk] = pltpu.async_copy(
                    x_hbm.at[x_slice(nk)], xbufs[jj], sl.at[jj])
            if b == B - 1 and c + 2 < NCH:
                pos_d[c + 2] = pltpu.async_copy(
                    pos_hbm.at[pl.ds(base + (c + 2) * _CH, _CH)],
                    pbufs[c % 2], sp.at[c % 2])

        for k in range(max(0, NX - _NBUF), NX):
            store_d[k].wait()

    return run(xf, pf).reshape(B, S, D)


# speedup vs baseline: 5.4658x; 3.0179x over previous
import functools

import jax
import jax.numpy as jnp
from jax import lax
from jax.experimental import pallas as pl
from jax.experimental.pallas import tpu as pltpu
from jax.experimental.pallas import tpu_sc as plsc

_INFO = plsc.get_sparse_core_info()
_NC = _INFO.num_cores
_NS = _INFO.num_subcores
_L = _INFO.num_lanes
_NW = _NC * _NS

_CR = 16
_NBUF = 5


def kernel(x, pos_table):
    B, S, D = x.shape
    RW = S // _NW
    NCH = RW // _CR
    NX = NCH * B
    NSL = _CR * D // _L
    CSL = D // _L

    @functools.partial(
        pl.kernel,
        mesh=plsc.VectorSubcoreMesh(core_axis_name="c", subcore_axis_name="s"),
        out_type=jax.ShapeDtypeStruct((B, S, D), jnp.float32),
        scratch_types=(
            [pltpu.VMEM((_CR, D), jnp.float32) for _ in range(2)]
            + [pltpu.VMEM((_CR, D), jnp.float32) for _ in range(_NBUF)]
            + [
                pltpu.SemaphoreType.DMA((2,)),
                pltpu.SemaphoreType.DMA((_NBUF,)),
                pltpu.SemaphoreType.DMA((_NBUF,)),
            ]
        ),
        compiler_params=pltpu.CompilerParams(use_tc_tiling_on_sc=True),
    )
    def run(x_hbm, pos_hbm, out_hbm, p0, p1, *rest):
        xbufs = list(rest[:_NBUF])
        sp, sl, ss = rest[_NBUF:]
        pbufs = [p0, p1]
        wid = lax.axis_index("s") * _NC + lax.axis_index("c")
        base = wid * RW

        pos_d, load_d, store_d = {}, {}, {}
        for c in range(min(2, NCH)):
            pos_d[c] = pltpu.async_copy(
                pos_hbm.at[pl.ds(base + c * _CR, _CR)], pbufs[c], sp.at[c])
        for k in range(min(_NBUF - 1, NX)):
            c, b = divmod(k, B)
            load_d[k] = pltpu.async_copy(
                x_hbm.at[b, pl.ds(base + c * _CR, _CR)],
                xbufs[k % _NBUF], sl.at[k % _NBUF])

        for k in range(NX):
            c, b = divmod(k, B)
            j = k % _NBUF
            if b == 0:
                pos_d[c].wait()
            load_d[k].wait()
            pb, xb = pbufs[c % 2], xbufs[j]

            @plsc.parallel_loop(0, NSL, unroll=8)
            def add_body(i, pb=pb, xb=xb):
                r = i // CSL
                col = pl.ds((i % CSL) * _L, _L)
                plsc.addupdate(xb.at[r, col], pb[r, col])

            store_d[k] = pltpu.async_copy(
                xb, out_hbm.at[b, pl.ds(base + c * _CR, _CR)], ss.at[j])
            nk = k + _NBUF - 1
            if nk < NX:
                nc, nb = divmod(nk, B)
                jj = nk % _NBUF
                if nk - _NBUF >= 0:
                    store_d[nk - _NBUF].wait()
                load_d[nk] = pltpu.async_copy(
                    x_hbm.at[nb, pl.ds(base + nc * _CR, _CR)],
                    xbufs[jj], sl.at[jj])
            if b == B - 1 and c + 2 < NCH:
                pos_d[c + 2] = pltpu.async_copy(
                    pos_hbm.at[pl.ds(base + (c + 2) * _CR, _CR)],
                    pbufs[c % 2], sp.at[c % 2])

        for k in range(max(0, NX - _NBUF), NX):
            store_d[k].wait()

    return run(x, pos_table[:S])
